# baseline (speedup 1.0000x reference)
import jax
import jax.numpy as jnp
from jax.experimental import pallas as pl

BLOCK_B = 4096


def _policy_body(x_ref, w_ref, b_ref, o_ref):
    logits = jnp.dot(x_ref[...], w_ref[...],
                     preferred_element_type=jnp.float32) + b_ref[...]
    m = jnp.max(logits, axis=-1, keepdims=True)
    e = jnp.exp(logits - m)
    o_ref[...] = e / jnp.sum(e, axis=-1, keepdims=True)


def kernel(state, W, b):
    B, D = state.shape
    A = W.shape[1]
    b2 = b.reshape(1, A)
    return pl.pallas_call(
        _policy_body,
        grid=(B // BLOCK_B,),
        in_specs=[
            pl.BlockSpec((BLOCK_B, D), lambda i: (i, 0)),
            pl.BlockSpec((D, A), lambda i: (0, 0)),
            pl.BlockSpec((1, A), lambda i: (0, 0)),
        ],
        out_specs=pl.BlockSpec((BLOCK_B, A), lambda i: (i, 0)),
        out_shape=jax.ShapeDtypeStruct((B, A), jnp.float32),
    )(state, W, b2)


# speedup vs baseline: 2.7057x; 2.7057x over previous
import jax
import jax.numpy as jnp
from jax.experimental import pallas as pl

BLOCK_B = 2048


def _policy_body(x_ref, w_ref, b_ref, o_ref):
    logits = jax.lax.dot_general(
        w_ref[...], x_ref[...],
        dimension_numbers=(((0,), (0,)), ((), ())),
        preferred_element_type=jnp.float32,
    ) + b_ref[...]
    m = jnp.max(logits, axis=0, keepdims=True)
    e = jnp.exp(logits - m)
    o_ref[...] = e / jnp.sum(e, axis=0, keepdims=True)


def kernel(state, W, b):
    B, D = state.shape
    A = W.shape[1]
    xT = state.T
    bc = b.reshape(A, 1)
    probsT = pl.pallas_call(
        _policy_body,
        grid=(B // BLOCK_B,),
        in_specs=[
            pl.BlockSpec((D, BLOCK_B), lambda i: (0, i)),
            pl.BlockSpec((D, A), lambda i: (0, 0)),
            pl.BlockSpec((A, 1), lambda i: (0, 0)),
        ],
        out_specs=pl.BlockSpec((A, BLOCK_B), lambda i: (0, i)),
        out_shape=jax.ShapeDtypeStruct((A, B), jnp.float32),
    )(xT, W, bc)
    return probsT.T


# speedup vs baseline: 2.7227x; 1.0063x over previous
import jax
import jax.numpy as jnp
from jax.experimental import pallas as pl

BLOCK_B = 2048


def _policy_body(x_ref, w_ref, b_ref, o_ref):
    logits = jax.lax.dot_general(
        w_ref[...], x_ref[...],
        dimension_numbers=(((0,), (0,)), ((), ())),
        preferred_element_type=jnp.float32,
    ) + b_ref[...]
    m = jnp.max(logits, axis=0, keepdims=True)
    e = jnp.exp(logits - m)
    r = 1.0 / jnp.sum(e, axis=0, keepdims=True)
    o_ref[...] = e * r


def kernel(state, W, b):
    B, D = state.shape
    A = W.shape[1]
    xT = state.T
    bc = b.reshape(A, 1)
    probsT = pl.pallas_call(
        _policy_body,
        grid=(B // BLOCK_B,),
        in_specs=[
            pl.BlockSpec((D, BLOCK_B), lambda i: (0, i)),
            pl.BlockSpec((D, A), lambda i: (0, 0)),
            pl.BlockSpec((A, 1), lambda i: (0, 0)),
        ],
        out_specs=pl.BlockSpec((A, BLOCK_B), lambda i: (0, i)),
        out_shape=jax.ShapeDtypeStruct((A, B), jnp.float32),
    )(xT, W, bc)
    return probsT.T
